# Initial kernel scaffold; baseline (speedup 1.0000x reference)
import jax
import jax.numpy as jnp
from jax.experimental import pallas as pl


def kernel(x, edge_index, batch, W0, b0, W1, b1, W2, b2, lin1_W, lin1_b, lin2_W, lin2_b):
    raise NotImplementedError("write your pallas kernel here")


# speedup vs baseline: 7.1085x; 7.1085x over previous
import functools

import jax
import jax.numpy as jnp
from jax import lax
from jax.experimental import pallas as pl
from jax.experimental.pallas import tpu as pltpu
from jax.experimental.pallas import tpu_sc as plsc

NC = 2
NS = 16
NW = NC * NS
CHUNK = 128
ROWS = 512


def _make_sc_msg(n_pad, d, e_pad):
    n_chunks = e_pad // (NW * CHUNK)
    stripe = n_pad // NS
    z_iters = stripe // CHUNK
    mesh = plsc.VectorSubcoreMesh(
        core_axis_name="c", subcore_axis_name="s", num_cores=NC, num_subcores=NS)

    @functools.partial(
        pl.kernel,
        out_type=jax.ShapeDtypeStruct((NC, n_pad, d), jnp.float32),
        mesh=mesh,
        scratch_types=[
            pltpu.VMEM((CHUNK,), jnp.int32),
            pltpu.VMEM((CHUNK,), jnp.int32),
            pltpu.VMEM((CHUNK, d), jnp.float32),
            pltpu.VMEM((CHUNK, d), jnp.float32),
            pltpu.VMEM_SHARED((n_pad, d), jnp.float32),
            pltpu.SemaphoreType.DMA,
        ],
    )
    def msg(src_hbm, dst_hbm, g_hbm, zeros_hbm, out_hbm,
            idx_s, idx_d, rows, zb, acc_sh, sem):
        c = lax.axis_index("c")
        t = lax.axis_index("s")
        wid = c * NS + t
        pltpu.sync_copy(zeros_hbm, zb)
        for k in range(z_iters):
            pltpu.sync_copy(zb, acc_sh.at[pl.ds(t * stripe + k * CHUNK, CHUNK)])
        plsc.subcore_barrier()

        base = wid * n_chunks * CHUNK

        def body(i, carry):
            off = base + i * CHUNK
            pltpu.sync_copy(src_hbm.at[pl.ds(off, CHUNK)], idx_s)
            pltpu.sync_copy(dst_hbm.at[pl.ds(off, CHUNK)], idx_d)
            pltpu.async_copy(g_hbm.at[idx_s], rows, sem).wait()
            pltpu.sync_copy(rows, acc_sh.at[idx_d], add=True)
            return carry

        lax.fori_loop(0, n_chunks, body, 0)
        plsc.subcore_barrier()
        for k in range(z_iters):
            off = t * stripe + k * CHUNK
            pltpu.sync_copy(acc_sh.at[pl.ds(off, CHUNK)],
                            out_hbm.at[c].at[pl.ds(off, CHUNK)])

    return msg


def _tc_pre(h, w, dinv, n_pad, d):
    def body(h_ref, w_ref, dinv_ref, o_ref):
        o_ref[...] = dinv_ref[...] * jnp.dot(
            h_ref[...], w_ref[...], preferred_element_type=jnp.float32)

    return pl.pallas_call(
        body,
        grid=(n_pad // ROWS,),
        in_specs=[
            pl.BlockSpec((ROWS, d), lambda i: (i, 0)),
            pl.BlockSpec((d, d), lambda i: (0, 0)),
            pl.BlockSpec((ROWS, 1), lambda i: (i, 0)),
        ],
        out_specs=pl.BlockSpec((ROWS, d), lambda i: (i, 0)),
        out_shape=jax.ShapeDtypeStruct((n_pad, d), jnp.float32),
    )(h, w, dinv)


def _tc_dinv(s_deg, n, n_pad, d):
    def body(s0_ref, s1_ref, o_ref):
        i = pl.program_id(0)
        deg = s0_ref[:, 0:1] + s1_ref[:, 0:1] + 1.0
        rid = i * ROWS + lax.broadcasted_iota(jnp.int32, (ROWS, 1), 0)
        o_ref[...] = jnp.where(rid < n, lax.rsqrt(deg), 0.0)

    return pl.pallas_call(
        body,
        grid=(n_pad // ROWS,),
        in_specs=[
            pl.BlockSpec((None, ROWS, d), lambda i: (0, i, 0)),
            pl.BlockSpec((None, ROWS, d), lambda i: (1, i, 0)),
        ],
        out_specs=pl.BlockSpec((ROWS, 1), lambda i: (i, 0)),
        out_shape=jax.ShapeDtypeStruct((n_pad, 1), jnp.float32),
    )(s_deg, s_deg)


def _tc_post(s_part, g, dinv, b, n, n_pad, d):
    def body(s0_ref, s1_ref, g_ref, dinv_ref, b_ref, o_ref):
        i = pl.program_id(0)
        t = dinv_ref[...] * (s0_ref[...] + s1_ref[...] + g_ref[...]) + b_ref[...]
        nrm = jnp.sqrt(jnp.sum(t * t, axis=1, keepdims=True))
        h = jnp.maximum(t / jnp.maximum(nrm, 1e-12), 0.0)
        rid = i * ROWS + lax.broadcasted_iota(jnp.int32, (ROWS, 1), 0)
        o_ref[...] = jnp.where(rid < n, h, 0.0)

    return pl.pallas_call(
        body,
        grid=(n_pad // ROWS,),
        in_specs=[
            pl.BlockSpec((None, ROWS, d), lambda i: (0, i, 0)),
            pl.BlockSpec((None, ROWS, d), lambda i: (1, i, 0)),
            pl.BlockSpec((ROWS, d), lambda i: (i, 0)),
            pl.BlockSpec((ROWS, 1), lambda i: (i, 0)),
            pl.BlockSpec((1, d), lambda i: (0, 0)),
        ],
        out_specs=pl.BlockSpec((ROWS, d), lambda i: (i, 0)),
        out_shape=jax.ShapeDtypeStruct((n_pad, d), jnp.float32),
    )(s_part, s_part, g, dinv, b)


def _tc_pool(h, n_pad, d):
    def body(h_ref, o_ref):
        i = pl.program_id(0)

        @pl.when(i == 0)
        def _():
            o_ref[...] = jnp.zeros_like(o_ref)

        o_ref[...] += jnp.sum(h_ref[...], axis=0, keepdims=True)

    return pl.pallas_call(
        body,
        grid=(n_pad // ROWS,),
        in_specs=[pl.BlockSpec((ROWS, d), lambda i: (i, 0))],
        out_specs=pl.BlockSpec((1, d), lambda i: (0, 0)),
        out_shape=jax.ShapeDtypeStruct((1, d), jnp.float32),
    )(h)


def _tc_head(pooled, w1, b1, w2p, b2p, d):
    def body(p_ref, w1_ref, b1_ref, w2_ref, b2_ref, o_ref):
        z = jnp.maximum(
            jnp.dot(p_ref[...], w1_ref[...], preferred_element_type=jnp.float32)
            + b1_ref[...], 0.0)
        o_ref[...] = jnp.dot(
            z, w2_ref[...], preferred_element_type=jnp.float32) + b2_ref[...]

    return pl.pallas_call(
        body,
        out_shape=jax.ShapeDtypeStruct((1, d), jnp.float32),
    )(pooled, w1, b1, w2p, b2p)


def kernel(x, edge_index, batch, W0, b0, W1, b1, W2, b2,
           lin1_W, lin1_b, lin2_W, lin2_b):
    n, d = x.shape
    e = edge_index.shape[1]
    c_out = lin2_W.shape[1]

    n_pad = ((n + 1 + 2047) // 2048) * 2048
    e_pad = ((e + NW * CHUNK - 1) // (NW * CHUNK)) * (NW * CHUNK)

    src = jnp.pad(edge_index[0].astype(jnp.int32), (0, e_pad - e),
                  constant_values=n)
    dst = jnp.pad(edge_index[1].astype(jnp.int32), (0, e_pad - e),
                  constant_values=n)
    x_pad = jnp.pad(x.astype(jnp.float32), ((0, n_pad - n), (0, 0)))
    zeros_blk = jnp.zeros((CHUNK, d), jnp.float32)
    ones_g = jnp.ones((n_pad, d), jnp.float32)

    sc_msg = _make_sc_msg(n_pad, d, e_pad)

    s_deg = sc_msg(src, dst, ones_g, zeros_blk)
    dinv = _tc_dinv(s_deg, n, n_pad, d)

    h = x_pad
    for (w, b) in ((W0, b0), (W1, b1), (W2, b2)):
        g = _tc_pre(h, w, dinv, n_pad, d)
        s_part = sc_msg(src, dst, g, zeros_blk)
        h = _tc_post(s_part, g, dinv, b.reshape(1, d), n, n_pad, d)

    pooled = _tc_pool(h, n_pad, d)

    w2p = jnp.pad(lin2_W, ((0, 0), (0, d - c_out)))
    b2p = jnp.pad(lin2_b, (0, d - c_out)).reshape(1, d)
    out_full = _tc_head(pooled, lin1_W, lin1_b.reshape(1, d), w2p, b2p, d)
    return out_full[:, :c_out]
